# baseline (speedup 1.0000x reference)
import functools

import jax
import jax.numpy as jnp
from jax import lax
from jax.experimental import pallas as pl

B, N, C = 16, 4096, 512
K = N // 2
T = 512
IT = N // T
KT = K // T


def _rank_body(scol_ref, srow_ref, ones_ref, rank_ref):
    it = pl.program_id(1)
    jt = pl.program_id(2)

    @pl.when(jt == 0)
    def _():
        rank_ref[...] = jnp.zeros_like(rank_ref)

    u_i = lax.bitcast_convert_type(scol_ref[0], jnp.int32) * 2
    u_j = lax.bitcast_convert_type(srow_ref[0], jnp.int32) * 2

    def count(thresh):
        cmp = thresh > u_i
        cnt = jnp.where(cmp, 1.0, 0.0)
        part = lax.dot_general(
            cnt, ones_ref[...], (((1,), (0,)), ((), ())),
            preferred_element_type=jnp.float32)
        rank_ref[0] += part

    @pl.when(it == jt)
    def _():
        ig = lax.broadcasted_iota(jnp.int32, (T, 1), 0)
        jg = lax.broadcasted_iota(jnp.int32, (1, T), 1)
        count(u_j + jnp.where(jg < ig, 1, 0))

    @pl.when(it != jt)
    def _():
        count(u_j + jnp.where(jt < it, 1, 0))


def _gather_body(rank_ref, s_ref, h_ref, out_ref):
    jt = pl.program_id(1)

    @pl.when(jt == 0)
    def _():
        out_ref[...] = jnp.zeros_like(out_ref)

    rank_row = rank_ref[...].reshape(1, T)
    s_row = s_ref[...].reshape(1, T)
    hmat = h_ref[0]
    for kt in range(KT):
        kio = jnp.asarray(kt * T, jnp.float32) + lax.broadcasted_iota(
            jnp.int32, (T, 1), 0).astype(jnp.float32)
        pmat = jnp.where(rank_row == kio, s_row, 0.0)
        out_ref[0, kt * T:(kt + 1) * T, :] += lax.dot_general(
            pmat, hmat, (((1,), (0,)), ((), ())),
            preferred_element_type=jnp.float32)


@jax.jit
def kernel(h, W, b):
    scores = jax.nn.sigmoid(jnp.einsum('bnc,oc->bno', h, W) + b)
    s_col = scores
    s_row = scores.reshape(B, 1, N)

    ones = jnp.ones((T, 1), jnp.float32)
    ranks = pl.pallas_call(
        _rank_body,
        grid=(B, IT, IT),
        in_specs=[
            pl.BlockSpec((1, T, 1), lambda b_, i, j: (b_, i, 0)),
            pl.BlockSpec((1, 1, T), lambda b_, i, j: (b_, 0, j)),
            pl.BlockSpec((T, 1), lambda b_, i, j: (0, 0)),
        ],
        out_specs=pl.BlockSpec((1, T, 1), lambda b_, i, j: (b_, i, 0)),
        out_shape=jax.ShapeDtypeStruct((B, N, 1), jnp.float32),
    )(s_col, s_row, ones)

    out = pl.pallas_call(
        _gather_body,
        grid=(B, IT),
        in_specs=[
            pl.BlockSpec((1, T, 1), lambda b_, j: (b_, j, 0)),
            pl.BlockSpec((1, 1, T), lambda b_, j: (b_, 0, j)),
            pl.BlockSpec((1, T, C), lambda b_, j: (b_, j, 0)),
        ],
        out_specs=pl.BlockSpec((1, K, C), lambda b_, j: (b_, 0, 0)),
        out_shape=jax.ShapeDtypeStruct((B, K, C), jnp.float32),
    )(ranks, s_row, h)
    return out


# speedup vs baseline: 17.2353x; 17.2353x over previous
import functools

import jax
import jax.numpy as jnp
from jax import lax
from jax.experimental import pallas as pl

B, N, C = 16, 4096, 512
K = N // 2
T = 512
IT = N // T
KT = K // T


def _rank_body(si_ref, sj_ref, rank_ref):
    it = pl.program_id(1)
    jt = pl.program_id(2)

    @pl.when(jt == 0)
    def _():
        rank_ref[...] = jnp.zeros_like(rank_ref)

    u_i = lax.bitcast_convert_type(
        si_ref[...].reshape(T, 1), jnp.int32) * 2
    u_j = lax.bitcast_convert_type(
        sj_ref[...].reshape(1, T), jnp.int32) * 2

    def count(thresh):
        cnt = (thresh > u_i).astype(jnp.int32)
        part = jnp.sum(cnt, axis=1, keepdims=True)
        rank_ref[...] += part.reshape(1, 1, T)

    @pl.when(it == jt)
    def _():
        ig = lax.broadcasted_iota(jnp.int32, (T, 1), 0)
        jg = lax.broadcasted_iota(jnp.int32, (1, T), 1)
        count(u_j + jnp.where(jg < ig, 1, 0))

    @pl.when(it != jt)
    def _():
        count(u_j + jnp.where(jt < it, 1, 0))


def _gather_body(rank_ref, s_ref, h_ref, out_ref):
    jt = pl.program_id(1)

    @pl.when(jt == 0)
    def _():
        out_ref[...] = jnp.zeros_like(out_ref)

    rank_row = rank_ref[...].reshape(1, T)
    s_row = s_ref[...].reshape(1, T)
    hmat = h_ref[0]
    for kt in range(KT):
        kio = kt * T + lax.broadcasted_iota(jnp.int32, (T, 1), 0)
        pmat = jnp.where(rank_row == kio, s_row, 0.0)
        out_ref[0, kt * T:(kt + 1) * T, :] += lax.dot_general(
            pmat, hmat, (((1,), (0,)), ((), ())),
            preferred_element_type=jnp.float32)


@jax.jit
def kernel(h, W, b):
    scores = jax.nn.sigmoid(jnp.einsum('bnc,oc->bno', h, W) + b)
    s_row = scores.reshape(B, 1, N)

    ranks = pl.pallas_call(
        _rank_body,
        grid=(B, IT, IT),
        in_specs=[
            pl.BlockSpec((1, 1, T), lambda b_, i, j: (b_, 0, i)),
            pl.BlockSpec((1, 1, T), lambda b_, i, j: (b_, 0, j)),
        ],
        out_specs=pl.BlockSpec((1, 1, T), lambda b_, i, j: (b_, 0, i)),
        out_shape=jax.ShapeDtypeStruct((B, 1, N), jnp.int32),
    )(s_row, s_row)

    out = pl.pallas_call(
        _gather_body,
        grid=(B, IT),
        in_specs=[
            pl.BlockSpec((1, 1, T), lambda b_, j: (b_, 0, j)),
            pl.BlockSpec((1, 1, T), lambda b_, j: (b_, 0, j)),
            pl.BlockSpec((1, T, C), lambda b_, j: (b_, j, 0)),
        ],
        out_specs=pl.BlockSpec((1, K, C), lambda b_, j: (b_, 0, 0)),
        out_shape=jax.ShapeDtypeStruct((B, K, C), jnp.float32),
    )(ranks, s_row, h)
    return out


# speedup vs baseline: 20.9925x; 1.2180x over previous
import functools

import jax
import jax.numpy as jnp
from jax import lax
from jax.experimental import pallas as pl

B, N, C = 16, 4096, 512
K = N // 2
T = 512
IT = N // T
KT = K // T


def _rank_body(si_ref, sj_ref, rank_ref):
    it = pl.program_id(1)
    jt = pl.program_id(2)

    @pl.when(jt == 0)
    def _():
        rank_ref[...] = jnp.zeros_like(rank_ref)

    u_i = lax.bitcast_convert_type(
        si_ref[...].reshape(1, T), jnp.int32) * 2
    u_j = lax.bitcast_convert_type(
        sj_ref[...].reshape(T, 1), jnp.int32) * 2

    def count(thresh):
        cnt = (thresh > u_i).astype(jnp.int32)
        part = jnp.sum(cnt, axis=0, keepdims=True)
        rank_ref[...] += part.reshape(1, 1, T)

    @pl.when(it == jt)
    def _():
        ig = lax.broadcasted_iota(jnp.int32, (1, T), 1)
        jg = lax.broadcasted_iota(jnp.int32, (T, 1), 0)
        count(u_j + jnp.where(jg < ig, 1, 0))

    @pl.when(it != jt)
    def _():
        count(u_j + jnp.where(jt < it, 1, 0))


def _gather_body(rank_ref, s_ref, h_ref, out_ref):
    jt = pl.program_id(1)

    @pl.when(jt == 0)
    def _():
        out_ref[...] = jnp.zeros_like(out_ref)

    rank_row = rank_ref[...].reshape(1, T)
    s_row = s_ref[...].reshape(1, T)
    hmat = h_ref[0]
    for kt in range(KT):
        kio = kt * T + lax.broadcasted_iota(jnp.int32, (T, 1), 0)
        pmat = jnp.where(rank_row == kio, s_row, 0.0)
        out_ref[0, kt * T:(kt + 1) * T, :] += lax.dot_general(
            pmat, hmat, (((1,), (0,)), ((), ())),
            preferred_element_type=jnp.float32)


@jax.jit
def kernel(h, W, b):
    scores = jax.nn.sigmoid(jnp.einsum('bnc,oc->bno', h, W) + b)
    s_row = scores.reshape(B, 1, N)

    ranks = pl.pallas_call(
        _rank_body,
        grid=(B, IT, IT),
        in_specs=[
            pl.BlockSpec((1, 1, T), lambda b_, i, j: (b_, 0, i)),
            pl.BlockSpec((1, 1, T), lambda b_, i, j: (b_, 0, j)),
        ],
        out_specs=pl.BlockSpec((1, 1, T), lambda b_, i, j: (b_, 0, i)),
        out_shape=jax.ShapeDtypeStruct((B, 1, N), jnp.int32),
    )(s_row, s_row)

    out = pl.pallas_call(
        _gather_body,
        grid=(B, IT),
        in_specs=[
            pl.BlockSpec((1, 1, T), lambda b_, j: (b_, 0, j)),
            pl.BlockSpec((1, 1, T), lambda b_, j: (b_, 0, j)),
            pl.BlockSpec((1, T, C), lambda b_, j: (b_, j, 0)),
        ],
        out_specs=pl.BlockSpec((1, K, C), lambda b_, j: (b_, 0, 0)),
        out_shape=jax.ShapeDtypeStruct((B, K, C), jnp.float32),
    )(ranks, s_row, h)
    return out
